# baseline (speedup 1.0000x reference)
import functools

import jax
import jax.numpy as jnp
from jax import lax
from jax.experimental import pallas as pl
from jax.experimental.pallas import tpu as pltpu
from jax.experimental.pallas import tpu_sc as plsc

LN_EPS = 1e-12

_NC = 2
_NS = 16
_NW = _NC * _NS

_GATHER_CHUNK = 256


def _sc_gather_body(table_hbm, idx_hbm, oidx_hbm, out_hbm,
                    idx_v0, idx_v1, oidx_v0, oidx_v1, rows_v0, rows_v1,
                    gsem0, gsem1, osem0, osem1):
    n_rows = idx_hbm.shape[0]
    per_w = n_rows // _NW
    wid = lax.axis_index("s") * _NC + lax.axis_index("c")
    base = wid * per_w
    c = _GATHER_CHUNK

    @pl.loop(0, per_w // c, step=2)
    def _(i):
        off0 = base + i * c
        off1 = off0 + c
        pltpu.sync_copy(idx_hbm.at[pl.ds(off0, c)], idx_v0)
        g0 = pltpu.async_copy(table_hbm.at[idx_v0], rows_v0, gsem0)
        pltpu.sync_copy(oidx_hbm.at[pl.ds(off0, c)], oidx_v0)
        pltpu.sync_copy(idx_hbm.at[pl.ds(off1, c)], idx_v1)
        g1 = pltpu.async_copy(table_hbm.at[idx_v1], rows_v1, gsem1)
        pltpu.sync_copy(oidx_hbm.at[pl.ds(off1, c)], oidx_v1)
        g0.wait()
        o0 = pltpu.async_copy(rows_v0, out_hbm.at[oidx_v0], osem0)
        g1.wait()
        o1 = pltpu.async_copy(rows_v1, out_hbm.at[oidx_v1], osem1)
        o0.wait()
        o1.wait()


def _sc_gather(table_pad, flat_ids, out_idx):
    n_rows = flat_ids.shape[0]
    dp = table_pad.shape[1]
    mesh = plsc.VectorSubcoreMesh(core_axis_name="c", subcore_axis_name="s")
    k = pl.kernel(
        _sc_gather_body,
        out_type=jax.ShapeDtypeStruct((n_rows, dp), table_pad.dtype),
        mesh=mesh,
        scratch_types=[
            pltpu.VMEM((_GATHER_CHUNK,), jnp.int32),
            pltpu.VMEM((_GATHER_CHUNK,), jnp.int32),
            pltpu.VMEM((_GATHER_CHUNK,), jnp.int32),
            pltpu.VMEM((_GATHER_CHUNK,), jnp.int32),
            pltpu.VMEM((_GATHER_CHUNK, dp), table_pad.dtype),
            pltpu.VMEM((_GATHER_CHUNK, dp), table_pad.dtype),
            pltpu.SemaphoreType.DMA,
            pltpu.SemaphoreType.DMA,
            pltpu.SemaphoreType.DMA,
            pltpu.SemaphoreType.DMA,
        ],
        compiler_params=pltpu.CompilerParams(use_tc_tiling_on_sc=False),
    )
    return k(table_pad, flat_ids, out_idx)


def _tc_body(xp_ref, g_ref, wt_ref, w_ref, b_ref, o_ref):
    x = xp_ref[...][:, 0:64]
    ids = g_ref[0]
    wt5 = wt_ref[...]

    r = x.shape[0]
    k_iota = lax.broadcasted_iota(jnp.int32, (5, r), 0)
    oh_t = (ids == k_iota).astype(jnp.float32)
    add = lax.dot_general(
        oh_t, wt5,
        dimension_numbers=(((0,), (0,)), ((), ())),
        preferred_element_type=jnp.float32,
    )
    x = x + add

    proj = jnp.full((64, 64), 1.0 / 64.0, jnp.float32)
    mu = lax.dot_general(
        x, proj,
        dimension_numbers=(((1,), (0,)), ((), ())),
        preferred_element_type=jnp.float32,
    )
    xc = x - mu
    var = lax.dot_general(
        xc * xc, proj,
        dimension_numbers=(((1,), (0,)), ((), ())),
        preferred_element_type=jnp.float32,
    )
    inv = lax.rsqrt(var + LN_EPS)
    y = xc * inv * w_ref[...] + b_ref[...]
    o_ref[0] = y.T


def _tc_add_ln(staging, ids3, wt5, ln_w, ln_b, s, b):
    return pl.pallas_call(
        _tc_body,
        grid=(s,),
        in_specs=[
            pl.BlockSpec((b, 128), lambda i: (i, 0)),
            pl.BlockSpec((1, 1, b), lambda i: (i, 0, 0)),
            pl.BlockSpec((5, 64), lambda i: (0, 0)),
            pl.BlockSpec((1, 64), lambda i: (0, 0)),
            pl.BlockSpec((1, 64), lambda i: (0, 0)),
        ],
        out_specs=pl.BlockSpec((1, 64, b), lambda i: (i, 0, 0)),
        out_shape=jax.ShapeDtypeStruct((s, 64, b), jnp.float32),
    )(staging, ids3, wt5, ln_w, ln_b)


def kernel(input_ids, gene_ids, gene_table, word_table, ln_weight, ln_bias):
    b, s = input_ids.shape
    v, d = gene_table.shape
    n_rows = b * s

    table_pad = jnp.pad(gene_table, ((0, 0), (0, 128 - d)))

    flat_ids = input_ids.reshape(n_rows).astype(jnp.int32)
    bb = lax.broadcasted_iota(jnp.int32, (b, s), 0)
    ss = lax.broadcasted_iota(jnp.int32, (b, s), 1)
    out_idx = (ss * b + bb).reshape(n_rows)

    staging = _sc_gather(table_pad, flat_ids, out_idx)

    g_full = jnp.concatenate(
        [jnp.full((b, 1), 4, jnp.int32), gene_ids.astype(jnp.int32)], axis=1
    )
    ids3 = g_full.T.reshape(s, 1, b)

    wt5 = jnp.concatenate(
        [word_table, jnp.zeros((1, d), word_table.dtype)], axis=0
    )

    out3 = _tc_add_ln(
        staging, ids3, wt5,
        ln_weight.reshape(1, d), ln_bias.reshape(1, d), s, b,
    )
    return jnp.transpose(out3, (2, 0, 1))


# speedup vs baseline: 1.1663x; 1.1663x over previous
import functools

import jax
import jax.numpy as jnp
from jax import lax
from jax.experimental import pallas as pl
from jax.experimental.pallas import tpu as pltpu
from jax.experimental.pallas import tpu_sc as plsc

LN_EPS = 1e-12

_NC = 2
_NS = 16
_NW = _NC * _NS

_GATHER_CHUNK = 512


def _sc_gather_body(table_hbm, idx_hbm, oidx_hbm, out_hbm,
                    idx_v0, idx_v1, oidx_v0, oidx_v1, rows_v0, rows_v1,
                    gsem0, gsem1, osem0, osem1):
    n_rows = idx_hbm.shape[0]
    per_w = n_rows // _NW
    wid = lax.axis_index("s") * _NC + lax.axis_index("c")
    base = wid * per_w
    c = _GATHER_CHUNK

    @pl.loop(0, per_w // c, step=2)
    def _(i):
        off0 = base + i * c
        off1 = off0 + c
        pltpu.sync_copy(idx_hbm.at[pl.ds(off0, c)], idx_v0)
        g0 = pltpu.async_copy(table_hbm.at[idx_v0], rows_v0, gsem0)
        pltpu.sync_copy(oidx_hbm.at[pl.ds(off0, c)], oidx_v0)
        pltpu.sync_copy(idx_hbm.at[pl.ds(off1, c)], idx_v1)
        g1 = pltpu.async_copy(table_hbm.at[idx_v1], rows_v1, gsem1)
        pltpu.sync_copy(oidx_hbm.at[pl.ds(off1, c)], oidx_v1)
        g0.wait()
        o0 = pltpu.async_copy(rows_v0, out_hbm.at[oidx_v0], osem0)
        g1.wait()
        o1 = pltpu.async_copy(rows_v1, out_hbm.at[oidx_v1], osem1)
        o0.wait()
        o1.wait()


def _sc_gather(table, flat_ids, out_idx):
    n_rows = flat_ids.shape[0]
    d = table.shape[1]
    mesh = plsc.VectorSubcoreMesh(core_axis_name="c", subcore_axis_name="s")
    k = pl.kernel(
        _sc_gather_body,
        out_type=jax.ShapeDtypeStruct((n_rows, d), table.dtype),
        mesh=mesh,
        scratch_types=[
            pltpu.VMEM((_GATHER_CHUNK,), jnp.int32),
            pltpu.VMEM((_GATHER_CHUNK,), jnp.int32),
            pltpu.VMEM((_GATHER_CHUNK,), jnp.int32),
            pltpu.VMEM((_GATHER_CHUNK,), jnp.int32),
            pltpu.VMEM((_GATHER_CHUNK, d), table.dtype),
            pltpu.VMEM((_GATHER_CHUNK, d), table.dtype),
            pltpu.SemaphoreType.DMA,
            pltpu.SemaphoreType.DMA,
            pltpu.SemaphoreType.DMA,
            pltpu.SemaphoreType.DMA,
        ],
        compiler_params=pltpu.CompilerParams(use_tc_tiling_on_sc=False),
    )
    return k(table, flat_ids, out_idx)


def _tc_body(xp_ref, g_ref, wt_ref, w_ref, b_ref, o_ref):
    xp = xp_ref[...]
    pid = g_ref[0]
    w25 = wt_ref[...]

    r2 = xp.shape[0]
    k_iota = lax.broadcasted_iota(jnp.int32, (25, r2), 0)
    oh_t = (pid == k_iota).astype(jnp.float32)
    add = lax.dot_general(
        oh_t, w25,
        dimension_numbers=(((0,), (0,)), ((), ())),
        preferred_element_type=jnp.float32,
    )
    x = xp + add

    ri = lax.broadcasted_iota(jnp.int32, (128, 128), 0)
    ci = lax.broadcasted_iota(jnp.int32, (128, 128), 1)
    proj = jnp.where((ri < 64) == (ci < 64), 1.0 / 64.0, 0.0)
    mu = lax.dot_general(
        x, proj,
        dimension_numbers=(((1,), (0,)), ((), ())),
        preferred_element_type=jnp.float32,
    )
    xc = x - mu
    var = lax.dot_general(
        xc * xc, proj,
        dimension_numbers=(((1,), (0,)), ((), ())),
        preferred_element_type=jnp.float32,
    )
    inv = lax.rsqrt(var + LN_EPS)
    y = xc * inv * w_ref[...] + b_ref[...]

    yt = y.T
    o_ref[0] = jnp.concatenate([yt[0:64, :], yt[64:128, :]], axis=1)


def _tc_add_ln(packed, pair_ids3, w25, ln_w2, ln_b2, s, b):
    r2 = b // 2
    return pl.pallas_call(
        _tc_body,
        grid=(s,),
        in_specs=[
            pl.BlockSpec((r2, 128), lambda i: (i, 0)),
            pl.BlockSpec((1, 1, r2), lambda i: (i, 0, 0)),
            pl.BlockSpec((25, 128), lambda i: (0, 0)),
            pl.BlockSpec((1, 128), lambda i: (0, 0)),
            pl.BlockSpec((1, 128), lambda i: (0, 0)),
        ],
        out_specs=pl.BlockSpec((1, 64, b), lambda i: (i, 0, 0)),
        out_shape=jax.ShapeDtypeStruct((s, 64, b), jnp.float32),
    )(packed, pair_ids3, w25, ln_w2, ln_b2)


def kernel(input_ids, gene_ids, gene_table, word_table, ln_weight, ln_bias):
    b, s = input_ids.shape
    v, d = gene_table.shape
    n_rows = b * s
    h = b // 2

    flat_ids = input_ids.reshape(n_rows).astype(jnp.int32)
    bb = lax.broadcasted_iota(jnp.int32, (b, s), 0)
    ss = lax.broadcasted_iota(jnp.int32, (b, s), 1)
    out_idx = (ss * b + lax.rem(bb, h) * 2 + bb // h).reshape(n_rows)

    gathered = _sc_gather(gene_table, flat_ids, out_idx)
    packed = gathered.reshape(n_rows // 2, 128)

    g_full = jnp.concatenate(
        [jnp.full((b, 1), 4, jnp.int32), gene_ids.astype(jnp.int32)], axis=1
    )
    pair_ids3 = (g_full[:h] * 5 + g_full[h:]).T.reshape(s, 1, h)

    wt5 = jnp.concatenate(
        [word_table, jnp.zeros((1, d), word_table.dtype)], axis=0
    )
    a_idx = jnp.arange(25) // 5
    b_idx = jnp.arange(25) % 5
    w25 = jnp.concatenate([wt5[a_idx], wt5[b_idx]], axis=1)

    ln_w2 = jnp.concatenate([ln_weight, ln_weight]).reshape(1, 2 * d)
    ln_b2 = jnp.concatenate([ln_bias, ln_bias]).reshape(1, 2 * d)

    out3 = _tc_add_ln(packed, pair_ids3, w25, ln_w2, ln_b2, s, b)
    return jnp.transpose(out3, (2, 0, 1))


# speedup vs baseline: 1.2112x; 1.0385x over previous
import functools

import jax
import jax.numpy as jnp
from jax import lax
from jax.experimental import pallas as pl
from jax.experimental.pallas import tpu as pltpu
from jax.experimental.pallas import tpu_sc as plsc

LN_EPS = 1e-12

_NC = 2
_NS = 16
_NW = _NC * _NS

_GATHER_CHUNK = 640


def _sc_gather_body(table_hbm, idx_hbm, oidx_hbm, out_hbm,
                    idx_v0, idx_v1, oidx_v0, oidx_v1, rows_v0, rows_v1,
                    gsem0, gsem1, osem0, osem1):
    n_rows = idx_hbm.shape[0]
    per_w = n_rows // _NW
    wid = lax.axis_index("s") * _NC + lax.axis_index("c")
    base = wid * per_w
    c = _GATHER_CHUNK

    @pl.loop(0, per_w // c, step=2)
    def _(i):
        off0 = base + i * c
        off1 = off0 + c
        pltpu.sync_copy(idx_hbm.at[pl.ds(off0, c)], idx_v0)
        g0 = pltpu.async_copy(table_hbm.at[idx_v0], rows_v0, gsem0)
        pltpu.sync_copy(oidx_hbm.at[pl.ds(off0, c)], oidx_v0)
        pltpu.sync_copy(idx_hbm.at[pl.ds(off1, c)], idx_v1)
        g1 = pltpu.async_copy(table_hbm.at[idx_v1], rows_v1, gsem1)
        pltpu.sync_copy(oidx_hbm.at[pl.ds(off1, c)], oidx_v1)
        g0.wait()
        o0 = pltpu.async_copy(rows_v0, out_hbm.at[oidx_v0], osem0)
        g1.wait()
        o1 = pltpu.async_copy(rows_v1, out_hbm.at[oidx_v1], osem1)
        o0.wait()
        o1.wait()


def _sc_gather(table, flat_ids, out_idx):
    n_rows = flat_ids.shape[0]
    d = table.shape[1]
    mesh = plsc.VectorSubcoreMesh(core_axis_name="c", subcore_axis_name="s")
    k = pl.kernel(
        _sc_gather_body,
        out_type=jax.ShapeDtypeStruct((n_rows, d), table.dtype),
        mesh=mesh,
        scratch_types=[
            pltpu.VMEM((_GATHER_CHUNK,), jnp.int32),
            pltpu.VMEM((_GATHER_CHUNK,), jnp.int32),
            pltpu.VMEM((_GATHER_CHUNK,), jnp.int32),
            pltpu.VMEM((_GATHER_CHUNK,), jnp.int32),
            pltpu.VMEM((_GATHER_CHUNK, d), table.dtype),
            pltpu.VMEM((_GATHER_CHUNK, d), table.dtype),
            pltpu.SemaphoreType.DMA,
            pltpu.SemaphoreType.DMA,
            pltpu.SemaphoreType.DMA,
            pltpu.SemaphoreType.DMA,
        ],
        compiler_params=pltpu.CompilerParams(use_tc_tiling_on_sc=False),
    )
    return k(table, flat_ids, out_idx)


def _tc_body(xp_ref, g_ref, wt_ref, w_ref, b_ref, o_ref):
    xp = xp_ref[...]
    pid = g_ref[0]
    w25 = wt_ref[...]

    r2 = xp.shape[0]
    k_iota = lax.broadcasted_iota(jnp.int32, (25, r2), 0)
    oh_t = (pid == k_iota).astype(jnp.float32)
    add = lax.dot_general(
        oh_t, w25,
        dimension_numbers=(((0,), (0,)), ((), ())),
        preferred_element_type=jnp.float32,
    )
    x = xp + add

    ri = lax.broadcasted_iota(jnp.int32, (128, 128), 0)
    ci = lax.broadcasted_iota(jnp.int32, (128, 128), 1)
    proj = jnp.where((ri < 64) == (ci < 64), 1.0 / 64.0, 0.0)
    mu = lax.dot_general(
        x, proj,
        dimension_numbers=(((1,), (0,)), ((), ())),
        preferred_element_type=jnp.float32,
    )
    xc = x - mu
    var = lax.dot_general(
        xc * xc, proj,
        dimension_numbers=(((1,), (0,)), ((), ())),
        preferred_element_type=jnp.float32,
    )
    inv = lax.rsqrt(var + LN_EPS)
    y = xc * inv * w_ref[...] + b_ref[...]

    yt = y.T
    o_ref[0] = jnp.concatenate([yt[0:64, :], yt[64:128, :]], axis=1)


def _tc_body_aliased(xp_ref, g_ref, wt_ref, w_ref, b_ref, _prev_ref, o_ref):
    _tc_body(xp_ref, g_ref, wt_ref, w_ref, b_ref, o_ref)


def _tc_add_ln_part(packed, pair_ids3, w25, ln_w2, ln_b2,
                    s_total, s_off, n_pos, b, out_prev=None):
    r2 = b // 2
    in_specs = [
        pl.BlockSpec((r2, 128), lambda i: (i, 0)),
        pl.BlockSpec((1, 1, r2), lambda i: (i + s_off, 0, 0)),
        pl.BlockSpec((25, 128), lambda i: (0, 0)),
        pl.BlockSpec((1, 128), lambda i: (0, 0)),
        pl.BlockSpec((1, 128), lambda i: (0, 0)),
    ]
    args = [packed, pair_ids3, w25, ln_w2, ln_b2]
    kwargs = {}
    body = _tc_body
    if out_prev is not None:
        in_specs.append(pl.BlockSpec(memory_space=pl.ANY))
        args.append(out_prev)
        kwargs["input_output_aliases"] = {5: 0}
        body = _tc_body_aliased
    return pl.pallas_call(
        body,
        grid=(n_pos,),
        in_specs=in_specs,
        out_specs=pl.BlockSpec((1, 64, b), lambda i: (i + s_off, 0, 0)),
        out_shape=jax.ShapeDtypeStruct((s_total, 64, b), jnp.float32),
        **kwargs,
    )(*args)


def kernel(input_ids, gene_ids, gene_table, word_table, ln_weight, ln_bias):
    b, s = input_ids.shape
    v, d = gene_table.shape
    h = b // 2
    sh = s // 2

    g_full = jnp.concatenate(
        [jnp.full((b, 1), 4, jnp.int32), gene_ids.astype(jnp.int32)], axis=1
    )
    pair_ids3 = (g_full[:h] * 5 + g_full[h:]).T.reshape(s, 1, h)

    wt5 = jnp.concatenate(
        [word_table, jnp.zeros((1, d), word_table.dtype)], axis=0
    )
    a_idx = jnp.arange(25) // 5
    b_idx = jnp.arange(25) % 5
    w25 = jnp.concatenate([wt5[a_idx], wt5[b_idx]], axis=1)

    ln_w2 = jnp.concatenate([ln_weight, ln_weight]).reshape(1, 2 * d)
    ln_b2 = jnp.concatenate([ln_bias, ln_bias]).reshape(1, 2 * d)

    bb = lax.broadcasted_iota(jnp.int32, (b, sh), 0)
    ssl = lax.broadcasted_iota(jnp.int32, (b, sh), 1)
    oidx_half = (ssl * b + lax.rem(bb, h) * 2 + bb // h).reshape(b * sh)

    packed_halves = []
    for k in range(2):
        flat_k = input_ids[:, k * sh:(k + 1) * sh].reshape(b * sh)
        staging_k = _sc_gather(gene_table, flat_k.astype(jnp.int32), oidx_half)
        packed_halves.append(staging_k.reshape(b * sh // 2, 128))

    out3 = _tc_add_ln_part(
        packed_halves[0], pair_ids3, w25, ln_w2, ln_b2, s, 0, sh, b)
    out3 = _tc_add_ln_part(
        packed_halves[1], pair_ids3, w25, ln_w2, ln_b2, s, sh, sh, b,
        out_prev=out3)
    return jnp.transpose(out3, (2, 0, 1))


# speedup vs baseline: 1.2891x; 1.0643x over previous
import functools

import jax
import jax.numpy as jnp
from jax import lax
from jax.experimental import pallas as pl
from jax.experimental.pallas import tpu as pltpu
from jax.experimental.pallas import tpu_sc as plsc

LN_EPS = 1e-12

_NC = 2
_NS = 16
_NW = _NC * _NS

_GATHER_CHUNK = 640


def _sc_gather_body(table_hbm, idx_hbm, oidx_hbm, out_hbm,
                    idx_v0, idx_v1, oidx_v0, oidx_v1, rows_v0, rows_v1,
                    gsem0, gsem1, osem0, osem1):
    n_rows = idx_hbm.shape[0]
    per_w = n_rows // _NW
    wid = lax.axis_index("s") * _NC + lax.axis_index("c")
    base = wid * per_w
    c = _GATHER_CHUNK

    @pl.loop(0, per_w // c, step=2)
    def _(i):
        off0 = base + i * c
        off1 = off0 + c
        pltpu.sync_copy(idx_hbm.at[pl.ds(off0, c)], idx_v0)
        g0 = pltpu.async_copy(table_hbm.at[idx_v0], rows_v0, gsem0)
        pltpu.sync_copy(oidx_hbm.at[pl.ds(off0, c)], oidx_v0)
        pltpu.sync_copy(idx_hbm.at[pl.ds(off1, c)], idx_v1)
        g1 = pltpu.async_copy(table_hbm.at[idx_v1], rows_v1, gsem1)
        pltpu.sync_copy(oidx_hbm.at[pl.ds(off1, c)], oidx_v1)
        g0.wait()
        o0 = pltpu.async_copy(rows_v0, out_hbm.at[oidx_v0], osem0)
        g1.wait()
        o1 = pltpu.async_copy(rows_v1, out_hbm.at[oidx_v1], osem1)
        o0.wait()
        o1.wait()


def _sc_gather(table, flat_ids, out_idx):
    n_rows = flat_ids.shape[0]
    d = table.shape[1]
    mesh = plsc.VectorSubcoreMesh(core_axis_name="c", subcore_axis_name="s")
    k = pl.kernel(
        _sc_gather_body,
        out_type=jax.ShapeDtypeStruct((n_rows, d), table.dtype),
        mesh=mesh,
        scratch_types=[
            pltpu.VMEM((_GATHER_CHUNK,), jnp.int32),
            pltpu.VMEM((_GATHER_CHUNK,), jnp.int32),
            pltpu.VMEM((_GATHER_CHUNK,), jnp.int32),
            pltpu.VMEM((_GATHER_CHUNK,), jnp.int32),
            pltpu.VMEM((_GATHER_CHUNK, d), table.dtype),
            pltpu.VMEM((_GATHER_CHUNK, d), table.dtype),
            pltpu.SemaphoreType.DMA,
            pltpu.SemaphoreType.DMA,
            pltpu.SemaphoreType.DMA,
            pltpu.SemaphoreType.DMA,
        ],
        compiler_params=pltpu.CompilerParams(use_tc_tiling_on_sc=False),
    )
    return k(table, flat_ids, out_idx)


def _tc_body(xp_ref, g_ref, wt_ref, w_ref, b_ref, o_ref):
    xp = xp_ref[...]
    pid = jnp.concatenate([g_ref[0], g_ref[1]], axis=1)
    w25 = wt_ref[...]

    r2 = xp.shape[0]
    k_iota = lax.broadcasted_iota(jnp.int32, (25, r2), 0)
    oh_t = (pid == k_iota).astype(jnp.float32)
    add = lax.dot_general(
        oh_t, w25,
        dimension_numbers=(((0,), (0,)), ((), ())),
        preferred_element_type=jnp.float32,
    )
    x = xp + add

    ri = lax.broadcasted_iota(jnp.int32, (128, 128), 0)
    ci = lax.broadcasted_iota(jnp.int32, (128, 128), 1)
    proj = jnp.where((ri < 64) == (ci < 64), 1.0 / 64.0, 0.0)
    mu = lax.dot_general(
        x, proj,
        dimension_numbers=(((1,), (0,)), ((), ())),
        preferred_element_type=jnp.float32,
    )
    xc = x - mu
    var = lax.dot_general(
        xc * xc, proj,
        dimension_numbers=(((1,), (0,)), ((), ())),
        preferred_element_type=jnp.float32,
    )
    inv = lax.rsqrt(var + LN_EPS)
    y = xc * inv * w_ref[...] + b_ref[...]

    yt = y.T
    half = r2 // 2
    o_ref[0] = jnp.concatenate(
        [yt[0:64, 0:half], yt[64:128, 0:half]], axis=1)
    o_ref[1] = jnp.concatenate(
        [yt[0:64, half:r2], yt[64:128, half:r2]], axis=1)


def _tc_body_aliased(xp_ref, g_ref, wt_ref, w_ref, b_ref, _prev_ref, o_ref):
    _tc_body(xp_ref, g_ref, wt_ref, w_ref, b_ref, o_ref)


def _tc_add_ln_part(packed, pair_ids3, w25, ln_w2, ln_b2,
                    s_total, s_off, n_pos, b, out_prev=None):
    r2 = b // 2
    so2 = s_off // 2
    in_specs = [
        pl.BlockSpec((2 * r2, 128), lambda i: (i, 0)),
        pl.BlockSpec((2, 1, r2), lambda i: (i + so2, 0, 0)),
        pl.BlockSpec((25, 128), lambda i: (0, 0)),
        pl.BlockSpec((1, 128), lambda i: (0, 0)),
        pl.BlockSpec((1, 128), lambda i: (0, 0)),
    ]
    args = [packed, pair_ids3, w25, ln_w2, ln_b2]
    kwargs = {}
    body = _tc_body
    if out_prev is not None:
        in_specs.append(pl.BlockSpec(memory_space=pl.ANY))
        args.append(out_prev)
        kwargs["input_output_aliases"] = {5: 0}
        body = _tc_body_aliased
    return pl.pallas_call(
        body,
        grid=(n_pos // 2,),
        in_specs=in_specs,
        out_specs=pl.BlockSpec((2, 64, b), lambda i: (i + so2, 0, 0)),
        out_shape=jax.ShapeDtypeStruct((s_total, 64, b), jnp.float32),
        **kwargs,
    )(*args)


def kernel(input_ids, gene_ids, gene_table, word_table, ln_weight, ln_bias):
    b, s = input_ids.shape
    v, d = gene_table.shape
    h = b // 2
    sh = s // 2

    g_full = jnp.concatenate(
        [jnp.full((b, 1), 4, jnp.int32), gene_ids.astype(jnp.int32)], axis=1
    )
    pair_ids3 = (g_full[:h] * 5 + g_full[h:]).T.reshape(s, 1, h)

    wt5 = jnp.concatenate(
        [word_table, jnp.zeros((1, d), word_table.dtype)], axis=0
    )
    a_idx = jnp.arange(25) // 5
    b_idx = jnp.arange(25) % 5
    w25 = jnp.concatenate([wt5[a_idx], wt5[b_idx]], axis=1)

    ln_w2 = jnp.concatenate([ln_weight, ln_weight]).reshape(1, 2 * d)
    ln_b2 = jnp.concatenate([ln_bias, ln_bias]).reshape(1, 2 * d)

    bb = lax.broadcasted_iota(jnp.int32, (b, sh), 0)
    ssl = lax.broadcasted_iota(jnp.int32, (b, sh), 1)
    oidx_half = (ssl * b + lax.rem(bb, h) * 2 + bb // h).reshape(b * sh)

    packed_halves = []
    for k in range(2):
        flat_k = input_ids[:, k * sh:(k + 1) * sh].reshape(b * sh)
        staging_k = _sc_gather(gene_table, flat_k.astype(jnp.int32), oidx_half)
        packed_halves.append(staging_k.reshape(b * sh // 2, 128))

    out3 = _tc_add_ln_part(
        packed_halves[0], pair_ids3, w25, ln_w2, ln_b2, s, 0, sh, b)
    out3 = _tc_add_ln_part(
        packed_halves[1], pair_ids3, w25, ln_w2, ln_b2, s, sh, sh, b,
        out_prev=out3)
    return jnp.transpose(out3, (2, 0, 1))


# speedup vs baseline: 1.3117x; 1.0176x over previous
import functools

import jax
import jax.numpy as jnp
from jax import lax
from jax.experimental import pallas as pl
from jax.experimental.pallas import tpu as pltpu
from jax.experimental.pallas import tpu_sc as plsc

LN_EPS = 1e-12

_NC = 2
_NS = 16
_NW = _NC * _NS

_GATHER_CHUNK = 640


def _sc_gather_body(table_hbm, idx_hbm, oidx_hbm, out_hbm,
                    idx_v0, idx_v1, oidx_v0, oidx_v1, rows_v0, rows_v1,
                    gsem0, gsem1, osem0, osem1):
    n_rows = idx_hbm.shape[0]
    per_w = n_rows // _NW
    wid = lax.axis_index("s") * _NC + lax.axis_index("c")
    base = wid * per_w
    c = _GATHER_CHUNK

    @pl.loop(0, per_w // c, step=2)
    def _(i):
        off0 = base + i * c
        off1 = off0 + c
        pltpu.sync_copy(idx_hbm.at[pl.ds(off0, c)], idx_v0)
        g0 = pltpu.async_copy(table_hbm.at[idx_v0], rows_v0, gsem0)
        pltpu.sync_copy(oidx_hbm.at[pl.ds(off0, c)], oidx_v0)
        pltpu.sync_copy(idx_hbm.at[pl.ds(off1, c)], idx_v1)
        g1 = pltpu.async_copy(table_hbm.at[idx_v1], rows_v1, gsem1)
        pltpu.sync_copy(oidx_hbm.at[pl.ds(off1, c)], oidx_v1)
        g0.wait()
        o0 = pltpu.async_copy(rows_v0, out_hbm.at[oidx_v0], osem0)
        g1.wait()
        o1 = pltpu.async_copy(rows_v1, out_hbm.at[oidx_v1], osem1)
        o0.wait()
        o1.wait()


def _sc_gather(table, flat_ids, out_idx):
    n_rows = flat_ids.shape[0]
    d = table.shape[1]
    mesh = plsc.VectorSubcoreMesh(core_axis_name="c", subcore_axis_name="s")
    k = pl.kernel(
        _sc_gather_body,
        out_type=jax.ShapeDtypeStruct((n_rows, d), table.dtype),
        mesh=mesh,
        scratch_types=[
            pltpu.VMEM((_GATHER_CHUNK,), jnp.int32),
            pltpu.VMEM((_GATHER_CHUNK,), jnp.int32),
            pltpu.VMEM((_GATHER_CHUNK,), jnp.int32),
            pltpu.VMEM((_GATHER_CHUNK,), jnp.int32),
            pltpu.VMEM((_GATHER_CHUNK, d), table.dtype),
            pltpu.VMEM((_GATHER_CHUNK, d), table.dtype),
            pltpu.SemaphoreType.DMA,
            pltpu.SemaphoreType.DMA,
            pltpu.SemaphoreType.DMA,
            pltpu.SemaphoreType.DMA,
        ],
        compiler_params=pltpu.CompilerParams(use_tc_tiling_on_sc=False),
    )
    return k(table, flat_ids, out_idx)


_POS_PER_STEP = 4


def _tc_body(xp_ref, g_ref, wt_ref, w_ref, b_ref, o_ref):
    xp = xp_ref[...]
    pid = jnp.concatenate(
        [g_ref[p] for p in range(_POS_PER_STEP)], axis=1)
    w25 = wt_ref[...]

    r2 = xp.shape[0]
    k_iota = lax.broadcasted_iota(jnp.int32, (25, r2), 0)
    oh_t = (pid == k_iota).astype(jnp.float32)
    add = lax.dot_general(
        oh_t, w25,
        dimension_numbers=(((0,), (0,)), ((), ())),
        preferred_element_type=jnp.float32,
    )
    x = xp + add

    ri = lax.broadcasted_iota(jnp.int32, (128, 128), 0)
    ci = lax.broadcasted_iota(jnp.int32, (128, 128), 1)
    proj = jnp.where((ri < 64) == (ci < 64), 1.0 / 64.0, 0.0)
    mu = lax.dot_general(
        x, proj,
        dimension_numbers=(((1,), (0,)), ((), ())),
        preferred_element_type=jnp.float32,
    )
    xc = x - mu
    var = lax.dot_general(
        xc * xc, proj,
        dimension_numbers=(((1,), (0,)), ((), ())),
        preferred_element_type=jnp.float32,
    )
    inv = lax.rsqrt(var + LN_EPS)
    y = xc * inv * w_ref[...] + b_ref[...]

    yt = y.T
    half = r2 // _POS_PER_STEP
    for p in range(_POS_PER_STEP):
        o_ref[p] = jnp.concatenate(
            [yt[0:64, p * half:(p + 1) * half],
             yt[64:128, p * half:(p + 1) * half]], axis=1)


def _tc_body_aliased(xp_ref, g_ref, wt_ref, w_ref, b_ref, _prev_ref, o_ref):
    _tc_body(xp_ref, g_ref, wt_ref, w_ref, b_ref, o_ref)


def _tc_add_ln_part(packed, pair_ids3, w25, ln_w2, ln_b2,
                    s_total, s_off, n_pos, b, out_prev=None):
    r2 = b // 2
    npos = _POS_PER_STEP
    so2 = s_off // npos
    in_specs = [
        pl.BlockSpec((npos * r2, 128), lambda i: (i, 0)),
        pl.BlockSpec((npos, 1, r2), lambda i: (i + so2, 0, 0)),
        pl.BlockSpec((25, 128), lambda i: (0, 0)),
        pl.BlockSpec((1, 128), lambda i: (0, 0)),
        pl.BlockSpec((1, 128), lambda i: (0, 0)),
    ]
    args = [packed, pair_ids3, w25, ln_w2, ln_b2]
    kwargs = {}
    body = _tc_body
    if out_prev is not None:
        in_specs.append(pl.BlockSpec(memory_space=pl.ANY))
        args.append(out_prev)
        kwargs["input_output_aliases"] = {5: 0}
        body = _tc_body_aliased
    return pl.pallas_call(
        body,
        grid=(n_pos // npos,),
        in_specs=in_specs,
        out_specs=pl.BlockSpec((npos, 64, b), lambda i: (i + so2, 0, 0)),
        out_shape=jax.ShapeDtypeStruct((s_total, 64, b), jnp.float32),
        **kwargs,
    )(*args)


def kernel(input_ids, gene_ids, gene_table, word_table, ln_weight, ln_bias):
    b, s = input_ids.shape
    v, d = gene_table.shape
    h = b // 2
    sh = s // 2

    g_full = jnp.concatenate(
        [jnp.full((b, 1), 4, jnp.int32), gene_ids.astype(jnp.int32)], axis=1
    )
    pair_ids3 = (g_full[:h] * 5 + g_full[h:]).T.reshape(s, 1, h)

    wt5 = jnp.concatenate(
        [word_table, jnp.zeros((1, d), word_table.dtype)], axis=0
    )
    a_idx = jnp.arange(25) // 5
    b_idx = jnp.arange(25) % 5
    w25 = jnp.concatenate([wt5[a_idx], wt5[b_idx]], axis=1)

    ln_w2 = jnp.concatenate([ln_weight, ln_weight]).reshape(1, 2 * d)
    ln_b2 = jnp.concatenate([ln_bias, ln_bias]).reshape(1, 2 * d)

    bb = lax.broadcasted_iota(jnp.int32, (b, sh), 0)
    ssl = lax.broadcasted_iota(jnp.int32, (b, sh), 1)
    oidx_half = (ssl * b + lax.rem(bb, h) * 2 + bb // h).reshape(b * sh)

    packed_halves = []
    for k in range(2):
        flat_k = input_ids[:, k * sh:(k + 1) * sh].reshape(b * sh)
        staging_k = _sc_gather(gene_table, flat_k.astype(jnp.int32), oidx_half)
        packed_halves.append(staging_k.reshape(b * sh // 2, 128))

    out3 = _tc_add_ln_part(
        packed_halves[0], pair_ids3, w25, ln_w2, ln_b2, s, 0, sh, b)
    out3 = _tc_add_ln_part(
        packed_halves[1], pair_ids3, w25, ln_w2, ln_b2, s, sh, sh, b,
        out_prev=out3)
    return jnp.transpose(out3, (2, 0, 1))
